# baseline (speedup 1.0000x reference)
import functools

import jax
import jax.numpy as jnp
from jax import lax
from jax.experimental import pallas as pl
from jax.experimental.pallas import tpu as pltpu
from jax.experimental.pallas import tpu_sc as plsc

N_WORKERS = 32
LANES = 16
D_MODEL = 64
SCALE = 8.0

V_TILES = 7813
PACK_ROWS = V_TILES * 64
PER_W1 = 246

_MESH = dict(core_axis_name="c", subcore_axis_name="s")
_TILED = pltpu.CompilerParams(
    use_tc_tiling_on_sc=True, needs_layout_passes=False)


def _iota16():
    return lax.iota(jnp.int32, 16)


@jax.jit
def _format_table(tt_t):
    mesh = plsc.VectorSubcoreMesh(**_MESH)

    @functools.partial(
        pl.kernel,
        out_type=jax.ShapeDtypeStruct((PACK_ROWS, 128), jnp.float32),
        mesh=mesh,
        scratch_types=(
            [pltpu.VMEM((D_MODEL, 128), jnp.float32)] * 4
            + [pltpu.SemaphoreType.DMA] * 4
        ),
        compiler_params=_TILED,
    )
    def k(tt_hbm, packed_hbm, iv0, iv1, ov0, ov1, is0, is1, ws0, ws1):
        ivs, ovs = (iv0, iv1), (ov0, ov1)
        isems, wsems = (is0, is1), (ws0, ws1)
        wid = lax.axis_index("s") * 2 + lax.axis_index("c")
        start = wid * PER_W1
        iota = _iota16()
        rowvecs = [iota + (c0 % D_MODEL) for c0 in range(0, 128, 16)]

        def vblk(u):
            return jnp.minimum(start + u, V_TILES - 1)

        def in_start(u, b):
            pltpu.async_copy(
                tt_hbm.at[:, pl.ds(vblk(u) * 128, 128)], ivs[b], isems[b])

        def in_wait(b):
            pltpu.make_async_copy(
                tt_hbm.at[:, pl.ds(0, 128)], ivs[b], isems[b]).wait()

        def out_start(u, b):
            pltpu.async_copy(
                ovs[b], packed_hbm.at[pl.ds(vblk(u) * 64, 64)], wsems[b])

        def out_wait(b):
            pltpu.make_async_copy(
                ovs[b], packed_hbm.at[pl.ds(0, 64)], wsems[b]).wait()

        def transpose(b):
            @plsc.parallel_loop(0, 64, unroll=2)
            def _(p):
                orow = ovs[b].at[p]
                for c0 in range(0, 128, 16):
                    par = 1 if c0 >= D_MODEL else 0
                    cols = jnp.full((16,), 2 * p + par, jnp.int32)
                    val = plsc.load_gather(ivs[b], [rowvecs[c0 // 16], cols])
                    orow[pl.ds(c0, 16)] = val

        in_start(0, 0)
        in_start(1, 1)

        @pl.loop(0, PER_W1, step=2)
        def _(u0):
            for j in range(2):
                u = u0 + j
                in_wait(j)

                @pl.when(u0 >= 2)
                def _():
                    out_wait(j)

                transpose(j)
                in_start(u + 2, j)
                out_start(u, j)

        in_wait(0)
        in_wait(1)
        out_wait(0)
        out_wait(1)

    return k(tt_t)


@jax.jit
def _embed(packed, x4, pos_t):
    tblks = x4.shape[0]
    mesh = plsc.VectorSubcoreMesh(**_MESH)

    @functools.partial(
        pl.kernel,
        out_type=jax.ShapeDtypeStruct((200, 8, 32, 8, 128), jnp.float32),
        mesh=mesh,
        scratch_types=(
            [pltpu.VMEM((D_MODEL, 256), jnp.float32)]
            + [pltpu.VMEM((8, 128), jnp.int32)] * 6
            + [pltpu.VMEM((128, 128), jnp.float32)] * 2
            + [pltpu.VMEM((D_MODEL, 128), jnp.float32)] * 2
            + [pltpu.SemaphoreType.DMA] * 6
        ),
        compiler_params=_TILED,
    )
    def k(packed_hbm, x4_hbm, pos_hbm, out_hbm,
          posv, ix0, ix1, j20, j21, pc0, pc1, rv0, rv1, ov0, ov1,
          xs0, xs1, gs0, gs1, ws0, ws1):
        ixs, ix2s, pcs = (ix0, ix1), (j20, j21), (pc0, pc1)
        rvs, ovs = (rv0, rv1), (ov0, ov1)
        xsems, gsems, wsems = (xs0, xs1), (gs0, gs1), (ws0, ws1)
        wid = lax.axis_index("s") * 2 + lax.axis_index("c")
        bblk = wid
        iota = _iota16()
        rowvecs = [iota + jb * 16 for jb in range(8)]

        pltpu.sync_copy(pos_hbm.at[:, pl.ds(0, 256)], posv)

        def ix_start(tb, b):
            pltpu.async_copy(
                x4_hbm.at[jnp.minimum(tb, tblks - 1), bblk], ixs[b], xsems[b])

        def ix_wait(b):
            pltpu.make_async_copy(
                x4_hbm.at[0, 0], ixs[b], xsems[b]).wait()

        def prep_idx(b):
            @pl.loop(0, 8)
            def _(ts):
                irow = ixs[b].at[ts]
                jrow = ix2s[b].at[ts]
                prow = pcs[b].at[ts]
                for jb in range(8):
                    sl = pl.ds(jb * 16, 16)
                    v = irow[sl]
                    jrow[sl] = v >> 1
                    prow[sl] = (v & 1) << 6

        def gather_start(ts, bt, br):
            pltpu.async_copy(
                packed_hbm.at[ix2s[bt].at[ts]], rvs[br], gsems[br])

        def gather_wait(br):
            pltpu.make_async_copy(
                packed_hbm.at[ix2s[0].at[0]], rvs[br], gsems[br]).wait()

        def wb_start(t, b):
            for dblk in range(8):
                pltpu.async_copy(
                    ovs[b].at[pl.ds(dblk * 8, 8)],
                    out_hbm.at[t, dblk, bblk], wsems[b])

        def wb_wait(b):
            for _ in range(8):
                pltpu.make_async_copy(
                    ovs[b].at[pl.ds(0, 8)], out_hbm.at[0, 0, bblk],
                    wsems[b]).wait()

        def compute(t, ts, bt, b):
            rv, ov = rvs[b], ovs[b]
            prow = pcs[bt].at[ts]
            par16s = [prow[pl.ds(jb * 16, 16)] for jb in range(8)]

            tvec = jnp.full((16,), t, jnp.int32)

            @plsc.parallel_loop(0, D_MODEL, unroll=2)
            def _(d):
                dvec = jnp.full((16,), d, jnp.int32)
                p = plsc.load_gather(posv, [dvec, tvec])
                orow = ov.at[d]
                for jb in range(8):
                    cols = par16s[jb] + d
                    val = plsc.load_gather(rv, [rowvecs[jb], cols])
                    orow[pl.ds(jb * 16, 16)] = val * SCALE + p

        ix_start(0, 0)
        ix_wait(0)
        prep_idx(0)
        ix_start(1, 1)
        gather_start(0, 0, 0)
        gather_start(1, 0, 1)

        def body(tb, bt, maybe_first):
            nxt = 1 - bt
            for ts in range(8):
                br = ts % 2
                gather_wait(br)
                if ts == 0:
                    ix_start(tb + 2, bt)
                if ts == 5:
                    ix_wait(nxt)
                    prep_idx(nxt)
                if maybe_first and ts < 2:
                    @pl.when(tb >= 1)
                    def _():
                        wb_wait(br)
                else:
                    wb_wait(br)
                t = tb * 8 + ts
                compute(t, ts, bt, br)
                if ts < 6:
                    gather_start(ts + 2, bt, br)
                else:
                    gather_start(ts - 6, nxt, br)
                wb_start(t, br)

        @pl.loop(0, tblks - 1, step=2)
        def _(tb0):
            body(tb0, 0, True)
            body(tb0 + 1, 1, False)

        body(jnp.int32(tblks - 1), 0, False)

        gather_wait(0)
        gather_wait(1)
        wb_wait(0)
        wb_wait(1)
        ix_wait(0)

    return k(packed, x4, pos_t)


def kernel(x, token_table, pos_table):
    b, t_cur = x.shape
    tt_t = token_table.T
    pos_t = pos_table.T
    x4 = (x.astype(jnp.int32).T
          .reshape(t_cur // 8, 8, b // 128, 128)
          .transpose(0, 2, 1, 3))
    packed = _format_table(tt_t)
    out5 = _embed(packed, x4, pos_t)
    return (out5.transpose(2, 4, 0, 1, 3)
            .reshape(b, t_cur, D_MODEL))


# speedup vs baseline: 1.5479x; 1.5479x over previous
import functools

import jax
import jax.numpy as jnp
from jax import lax
from jax.experimental import pallas as pl
from jax.experimental.pallas import tpu as pltpu
from jax.experimental.pallas import tpu_sc as plsc

N_WORKERS = 32
LANES = 16
D_MODEL = 64
SCALE = 8.0
PITCH = 137

V_TILES = 7813
V_PAD = V_TILES * 128
PER_W1 = 246

_MESH = dict(core_axis_name="c", subcore_axis_name="s")


def _iota16():
    return lax.iota(jnp.int32, 16)


@jax.jit
def _format_table(tt_t):
    mesh = plsc.VectorSubcoreMesh(**_MESH)

    @functools.partial(
        pl.kernel,
        out_type=jax.ShapeDtypeStruct((V_PAD // 2, 128), jnp.float32),
        mesh=mesh,
        scratch_types=(
            [pltpu.VMEM((D_MODEL, PITCH), jnp.float32)] * 2
            + [pltpu.VMEM((D_MODEL, 128), jnp.float32)] * 2
            + [pltpu.SemaphoreType.DMA] * 4
        ),
        compiler_params=pltpu.CompilerParams(
            use_tc_tiling_on_sc=True, needs_layout_passes=False),
    )
    def k(tt_hbm, packed_hbm, iv0, iv1, ov0, ov1, is0, is1, ws0, ws1):
        ivs, ovs = (iv0, iv1), (ov0, ov1)
        isems, wsems = (is0, is1), (ws0, ws1)
        wid = lax.axis_index("s") * 2 + lax.axis_index("c")
        start = wid * PER_W1
        iota = _iota16()
        rowvecs = [iota + (c0 % D_MODEL) for c0 in range(0, 128, 16)]

        def vblk(u):
            return jnp.minimum(start + u, V_TILES - 1)

        def in_start(u, b):
            pltpu.async_copy(
                tt_hbm.at[:, pl.ds(vblk(u) * 128, 128)],
                ivs[b].at[:, pl.ds(0, 128)], isems[b])

        def in_wait(b):
            pltpu.make_async_copy(
                tt_hbm.at[:, pl.ds(0, 128)],
                ivs[b].at[:, pl.ds(0, 128)], isems[b]).wait()

        def out_start(u, b):
            pltpu.async_copy(
                ovs[b], packed_hbm.at[pl.ds(vblk(u) * 64, 64)], wsems[b])

        def out_wait(b):
            pltpu.make_async_copy(
                ovs[b], packed_hbm.at[pl.ds(0, 64)], wsems[b]).wait()

        def transpose(b):
            @plsc.parallel_loop(0, 64, unroll=2)
            def _(p):
                orow = ovs[b].at[p]
                for c0 in range(0, 128, 16):
                    par = 1 if c0 >= D_MODEL else 0
                    cols = jnp.full((16,), 2 * p + par, jnp.int32)
                    val = plsc.load_gather(ivs[b], [rowvecs[c0 // 16], cols])
                    orow[pl.ds(c0, 16)] = val

        in_start(0, 0)
        in_start(1, 1)

        @pl.loop(0, PER_W1, step=2)
        def _(u0):
            for j in range(2):
                u = u0 + j
                in_wait(j)

                @pl.when(u0 >= 2)
                def _():
                    out_wait(j)

                transpose(j)
                in_start(u + 2, j)
                out_start(u, j)

        in_wait(0)
        in_wait(1)
        out_wait(0)
        out_wait(1)

    return k(tt_t)


@jax.jit
def _embed(rows_tab, x4, pos):
    tblks = x4.shape[0]

    mesh = plsc.VectorSubcoreMesh(**_MESH)

    @functools.partial(
        pl.kernel,
        out_type=jax.ShapeDtypeStruct((200, 8, 32, 8, 128), jnp.float32),
        mesh=mesh,
        scratch_types=(
            [pltpu.VMEM((200, D_MODEL), jnp.float32)]
            + [pltpu.VMEM((8, 128), jnp.int32)] * 2
            + [pltpu.VMEM((128, D_MODEL), jnp.float32)] * 2
            + [pltpu.VMEM((D_MODEL, PITCH), jnp.float32)] * 2
            + [pltpu.SemaphoreType.DMA] * 6
        ),
        compiler_params=pltpu.CompilerParams(
            use_tc_tiling_on_sc=False, needs_layout_passes=False),
    )
    def k(tab_hbm, x4_hbm, pos_hbm, out_hbm,
          posv, ix0, ix1, rv0, rv1, ov0, ov1,
          xs0, xs1, gs0, gs1, ws0, ws1):
        ixs = (ix0, ix1)
        rvs, ovs = (rv0, rv1), (ov0, ov1)
        xsems, gsems, wsems = (xs0, xs1), (gs0, gs1), (ws0, ws1)
        wid = lax.axis_index("s") * 2 + lax.axis_index("c")
        bblk = wid
        iota = _iota16()
        rowbases = [iota + k16 for k16 in range(0, D_MODEL, 16)]

        pltpu.sync_copy(pos_hbm, posv)

        def ix_start(tb, b):
            pltpu.async_copy(
                x4_hbm.at[jnp.minimum(tb, tblks - 1), bblk], ixs[b], xsems[b])

        def ix_wait(b):
            pltpu.make_async_copy(x4_hbm.at[0, 0], ixs[b], xsems[b]).wait()

        def gather_start(ts, bt, br):
            pltpu.async_copy(
                tab_hbm.at[ixs[bt].at[ts]], rvs[br], gsems[br])

        def gather_wait(br):
            pltpu.make_async_copy(
                tab_hbm.at[ixs[0].at[0]], rvs[br], gsems[br]).wait()

        def wb_start(t, b):
            for dblk in range(8):
                pltpu.async_copy(
                    ovs[b].at[pl.ds(dblk * 8, 8), pl.ds(0, 128)],
                    out_hbm.at[t, dblk, bblk], wsems[b])

        def wb_wait(b):
            for _ in range(8):
                pltpu.make_async_copy(
                    ovs[b].at[pl.ds(0, 8), pl.ds(0, 128)],
                    out_hbm.at[0, 0, bblk], wsems[b]).wait()

        def compute(t, br):
            rv, ov = rvs[br], ovs[br]
            prow = posv.at[t]
            pvs = [prow[pl.ds(k16, 16)] for k16 in range(0, D_MODEL, 16)]

            @plsc.parallel_loop(0, 128, unroll=2)
            def _(j):
                rrow = rv.at[j]
                jv = jnp.full((16,), j, jnp.int32)
                for kk in range(D_MODEL // 16):
                    val = rrow[pl.ds(kk * 16, 16)]
                    plsc.store_scatter(
                        ov, [rowbases[kk], jv], val * SCALE + pvs[kk])

        ix_start(0, 0)
        ix_wait(0)
        gather_start(0, 0, 0)
        gather_start(1, 0, 1)

        def body(tb, bt, maybe_first):
            nxt = 1 - bt
            for ts in range(8):
                br = ts % 2
                gather_wait(br)
                if ts == 2:
                    ix_start(tb + 1, nxt)
                if ts == 5:
                    ix_wait(nxt)
                if maybe_first and ts < 2:
                    @pl.when(tb >= 1)
                    def _():
                        wb_wait(br)
                else:
                    wb_wait(br)
                t = tb * 8 + ts
                compute(t, br)
                if ts < 6:
                    gather_start(ts + 2, bt, br)
                else:
                    gather_start(ts - 6, nxt, br)
                wb_start(t, br)

        @pl.loop(0, tblks - 1, step=2)
        def _(tb0):
            body(tb0, 0, True)
            body(tb0 + 1, 1, False)

        body(jnp.int32(tblks - 1), 0, False)

        gather_wait(0)
        gather_wait(1)
        wb_wait(0)
        wb_wait(1)

    return k(rows_tab, x4, pos)


def kernel(x, token_table, pos_table):
    b, t_cur = x.shape
    tt_t = token_table.T
    x4 = (x.astype(jnp.int32).T
          .reshape(t_cur // 8, 8, b // 128, 128)
          .transpose(0, 2, 1, 3))
    packed = _format_table(tt_t)
    rows_tab = packed.reshape(V_PAD, D_MODEL)
    out5 = _embed(rows_tab, x4, pos_table[:t_cur])
    return (out5.transpose(2, 4, 0, 1, 3)
            .reshape(b, t_cur, D_MODEL))
